# baseline (speedup 1.0000x reference)
import functools
import math

import jax
import jax.numpy as jnp
from jax.experimental import pallas as pl
from jax.experimental.pallas import tpu as pltpu

_OUTLIER_RATIO = 0.08


def _f16_round(v):
    u = jax.lax.bitcast_convert_type(v, jnp.uint32)
    rn = (u + jnp.uint32(0x1000)) & jnp.uint32(0xFFFFE000)
    return jax.lax.bitcast_convert_type(rn, jnp.float32)


def _fused_kernel(x_ref, pi_ref, s_ref, cent_ref, bound_ref, out_ref,
                  sum_ref, sumsq_ref, mask_ref, norms_ref, inv_ref,
                  *, nblocks, n, k, scale):
    p = pl.program_id(0)
    i = pl.program_id(1)
    d = out_ref.shape[1]
    blk = out_ref.shape[0]

    @pl.when(p == 0)
    def _stats():
        xb = x_ref[...]
        xsq = xb * xb
        csum = jnp.sum(xb, axis=0, keepdims=True)
        csumsq = jnp.sum(xsq, axis=0, keepdims=True)
        norms = jnp.sqrt(jnp.sum(xsq, axis=1, keepdims=True))
        norms_ref[pl.ds(i * blk, blk), :] = norms
        inv_ref[pl.ds(i * blk, blk), :] = 1.0 / (norms + 1e-10)

        @pl.when(i == 0)
        def _():
            sum_ref[...] = csum
            sumsq_ref[...] = csumsq

        @pl.when(i > 0)
        def _():
            sum_ref[...] += csum
            sumsq_ref[...] += csumsq

        @pl.when(i == nblocks - 1)
        def _():
            mean = sum_ref[...] / n
            var = sumsq_ref[...] / n - mean * mean
            vcol = var.reshape(d, 1)
            jj = jax.lax.broadcasted_iota(jnp.int32, (d, d), 1)
            ii = jax.lax.broadcasted_iota(jnp.int32, (d, d), 0)
            before = (var > vcol) | ((var == vcol) & (jj < ii))
            rank = jnp.sum(before.astype(jnp.int32), axis=1, keepdims=True)
            mask_col = jnp.where(rank < k, 0.0, 1.0)
            mask_ref[...] = mask_col.reshape(1, d)

    @pl.when(p == 1)
    def _main():
        m = mask_ref[...]
        pi = pi_ref[...]
        s = s_ref[...]
        b0, b1, b2 = bound_ref[0], bound_ref[1], bound_ref[2]
        c0, c1, c2, c3 = cent_ref[0], cent_ref[1], cent_ref[2], cent_ref[3]

        def chain(lo, sub):
            xb = x_ref[pl.ds(lo, sub), :]
            xsq = xb * xb
            norms = norms_ref[pl.ds(i * blk + lo, sub), :]
            inv = inv_ref[pl.ds(i * blk + lo, sub), :]
            rn = jnp.sqrt(jnp.sum(xsq * m, axis=1, keepdims=True)) * inv
            invrn = 1.0 / (rn + 1e-10)
            xu = xb * inv
            xr = xu * m
            xru = xr * invrn
            y = jax.lax.dot_general(xru, pi, (((1,), (1,)), ((), ())),
                                    preferred_element_type=jnp.float32)
            yh = jnp.where(y > b1,
                           jnp.where(y > b2, c3, c2),
                           jnp.where(y > b0, c1, c0))
            xm = jax.lax.dot_general(yh, pi, (((1,), (0,)), ((), ())),
                                     preferred_element_type=jnp.float32) * rn
            r = xr - xm
            proj = jax.lax.dot_general(r, s, (((1,), (1,)), ((), ())),
                                       preferred_element_type=jnp.float32)
            sg = jnp.where(proj > 0, 1.0, -1.0)
            resn = jnp.sqrt(jnp.sum(r * r, axis=1, keepdims=True))
            rh = jax.lax.dot_general(sg, s, (((1,), (0,)), ((), ())),
                                     preferred_element_type=jnp.float32)
            xh = xm + rh * (scale * resn)
            pt = _f16_round(xu)
            out_ref[pl.ds(lo, sub), :] = jnp.where(m > 0.5, xh, pt) * norms

        sub = blk // 2
        chain(0, sub)
        chain(sub, sub)


def kernel(x, Pi, centroids, S, decision_boundaries):
    n, d = x.shape
    k = max(1, int(d * _OUTLIER_RATIO))
    scale = math.sqrt(math.pi / 2.0) / d

    blk = 1024
    nblocks = n // blk
    out = pl.pallas_call(
        functools.partial(_fused_kernel, nblocks=nblocks, n=float(n), k=k,
                          scale=scale),
        grid=(2, nblocks),
        in_specs=[
            pl.BlockSpec((blk, d), lambda p, i: (i, 0)),
            pl.BlockSpec((d, d), lambda p, i: (0, 0)),
            pl.BlockSpec((d, d), lambda p, i: (0, 0)),
            pl.BlockSpec(memory_space=pltpu.SMEM),
            pl.BlockSpec(memory_space=pltpu.SMEM),
        ],
        out_specs=pl.BlockSpec((blk, d), lambda p, i: (p * i, 0)),
        out_shape=jax.ShapeDtypeStruct((n, d), jnp.float32),
        scratch_shapes=[pltpu.VMEM((1, d), jnp.float32),
                        pltpu.VMEM((1, d), jnp.float32),
                        pltpu.VMEM((1, d), jnp.float32),
                        pltpu.VMEM((n, 1), jnp.float32),
                        pltpu.VMEM((n, 1), jnp.float32)],
    )(x, Pi, S, centroids, decision_boundaries)
    return out


# speedup vs baseline: 1.0890x; 1.0890x over previous
import functools
import math

import jax
import jax.numpy as jnp
from jax.experimental import pallas as pl
from jax.experimental.pallas import tpu as pltpu

_OUTLIER_RATIO = 0.08


def _f16_round(v):
    u = jax.lax.bitcast_convert_type(v, jnp.uint32)
    rn = (u + jnp.uint32(0x1000)) & jnp.uint32(0xFFFFE000)
    return jax.lax.bitcast_convert_type(rn, jnp.float32)


def _fused_kernel(x_ref, pi_ref, s_ref, cent_ref, bound_ref, out_ref,
                  sum_ref, sumsq_ref, mask_ref, *, nblocks, n, k, scale):
    p = pl.program_id(0)
    i = pl.program_id(1)
    d = out_ref.shape[1]
    blk = out_ref.shape[0]

    @pl.when(p == 0)
    def _stats():
        xb = x_ref[...]
        csum = jnp.sum(xb, axis=0, keepdims=True)
        csumsq = jnp.sum(xb * xb, axis=0, keepdims=True)

        @pl.when(i == 0)
        def _():
            sum_ref[...] = csum
            sumsq_ref[...] = csumsq

        @pl.when(i > 0)
        def _():
            sum_ref[...] += csum
            sumsq_ref[...] += csumsq

        @pl.when(i == nblocks - 1)
        def _():
            mean = sum_ref[...] / n
            var = sumsq_ref[...] / n - mean * mean
            vcol = var.reshape(d, 1)
            jj = jax.lax.broadcasted_iota(jnp.int32, (d, d), 1)
            ii = jax.lax.broadcasted_iota(jnp.int32, (d, d), 0)
            before = (var > vcol) | ((var == vcol) & (jj < ii))
            rank = jnp.sum(before.astype(jnp.int32), axis=1, keepdims=True)
            mask_col = jnp.where(rank < k, 0.0, 1.0)
            mask_ref[...] = mask_col.reshape(1, d)

    @pl.when(p == 1)
    def _main():
        m = mask_ref[...]
        pi = pi_ref[...]
        s = s_ref[...]
        b0, b1, b2 = bound_ref[0], bound_ref[1], bound_ref[2]
        c0, c1, c2, c3 = cent_ref[0], cent_ref[1], cent_ref[2], cent_ref[3]

        def chain(lo, sub):
            xb = x_ref[pl.ds(lo, sub), :]
            xsq = xb * xb
            norms = jnp.sqrt(jnp.sum(xsq, axis=1, keepdims=True))
            inv = 1.0 / (norms + 1e-10)
            rn = jnp.sqrt(jnp.sum(xsq * m, axis=1, keepdims=True)) * inv
            invrn = 1.0 / (rn + 1e-10)
            xu = xb * inv
            xr = xu * m
            xru = xr * invrn
            y = jax.lax.dot_general(xru, pi, (((1,), (1,)), ((), ())),
                                    preferred_element_type=jnp.float32)
            yh = jnp.where(y > b1,
                           jnp.where(y > b2, c3, c2),
                           jnp.where(y > b0, c1, c0))
            xm = jax.lax.dot_general(yh, pi, (((1,), (0,)), ((), ())),
                                     preferred_element_type=jnp.float32) * rn
            r = xr - xm
            proj = jax.lax.dot_general(r, s, (((1,), (1,)), ((), ())),
                                       preferred_element_type=jnp.float32)
            sg = jnp.where(proj > 0, 1.0, -1.0)
            resn = jnp.sqrt(jnp.sum(r * r, axis=1, keepdims=True))
            rh = jax.lax.dot_general(sg, s, (((1,), (0,)), ((), ())),
                                     preferred_element_type=jnp.float32)
            xh = xm + rh * (scale * resn)
            pt = _f16_round(xu)
            out_ref[pl.ds(lo, sub), :] = jnp.where(m > 0.5, xh, pt) * norms

        sub = 512
        for lo in range(0, blk, sub):
            chain(lo, sub)


def kernel(x, Pi, centroids, S, decision_boundaries):
    n, d = x.shape
    k = max(1, int(d * _OUTLIER_RATIO))
    scale = math.sqrt(math.pi / 2.0) / d

    blk = 2048
    nblocks = n // blk
    out = pl.pallas_call(
        functools.partial(_fused_kernel, nblocks=nblocks, n=float(n), k=k,
                          scale=scale),
        grid=(2, nblocks),
        in_specs=[
            pl.BlockSpec((blk, d), lambda p, i: (i, 0)),
            pl.BlockSpec((d, d), lambda p, i: (0, 0)),
            pl.BlockSpec((d, d), lambda p, i: (0, 0)),
            pl.BlockSpec(memory_space=pltpu.SMEM),
            pl.BlockSpec(memory_space=pltpu.SMEM),
        ],
        out_specs=pl.BlockSpec((blk, d), lambda p, i: (p * i, 0)),
        out_shape=jax.ShapeDtypeStruct((n, d), jnp.float32),
        scratch_shapes=[pltpu.VMEM((1, d), jnp.float32),
                        pltpu.VMEM((1, d), jnp.float32),
                        pltpu.VMEM((1, d), jnp.float32)],
    )(x, Pi, S, centroids, decision_boundaries)
    return out
